# initial kernel scaffold (unmeasured)
import jax
import jax.numpy as jnp
from jax import lax
from jax.experimental import pallas as pl
from jax.experimental.pallas import tpu as pltpu

N_DEV = 16
N_STAGES = 4
SQ = 256
HQ = 8
DH = 128
D = HQ * DH
SCALE = 0.08838834764831843


def kernel(x, Wq, Wo, K_ext, V_ext):
    def body(x_ref, wq_ref, wo_ref, k_ref, v_ref, out_ref,
             acc_ref, st_ref, rbuf_o, rbuf_st,
             o_send, o_recv, st_send, st_recv):
        my = lax.axis_index("i")

        barrier = pltpu.get_barrier_semaphore()
        for k in range(N_STAGES):
            partner = my ^ (1 << k)
            pl.semaphore_signal(barrier, inc=1, device_id=(partner,),
                                device_id_type=pl.DeviceIdType.MESH)
        pl.semaphore_wait(barrier, N_STAGES)

        q = jnp.dot(x_ref[0], wq_ref[...],
                    preferred_element_type=jnp.float32) * SCALE
        for h in range(HQ):
            qh = q[:, h * DH:(h + 1) * DH]
            kh = k_ref[0, :, h, :]
            vh = v_ref[0, :, h, :]
            s = lax.dot_general(qh, kh, (((1,), (1,)), ((), ())),
                                preferred_element_type=jnp.float32)
            m = jnp.max(s, axis=1, keepdims=True)
            p = jnp.exp(s - m)
            l = jnp.sum(p, axis=1, keepdims=True)
            o = jnp.dot(p, vh, preferred_element_type=jnp.float32)
            acc_ref[:, h * DH:(h + 1) * DH] = o
            st_ref[:, h:h + 1] = m
            st_ref[:, HQ + h:HQ + h + 1] = l

        for k in range(N_STAGES):
            partner = my ^ (1 << k)
            o_rdma = pltpu.make_async_remote_copy(
                src_ref=acc_ref, dst_ref=rbuf_o.at[k],
                send_sem=o_send.at[k], recv_sem=o_recv.at[k],
                device_id=(partner,), device_id_type=pl.DeviceIdType.MESH)
            st_rdma = pltpu.make_async_remote_copy(
                src_ref=st_ref, dst_ref=rbuf_st.at[k],
                send_sem=st_send.at[k], recv_sem=st_recv.at[k],
                device_id=(partner,), device_id_type=pl.DeviceIdType.MESH)
            o_rdma.start()
            st_rdma.start()
            o_rdma.wait()
            st_rdma.wait()

            m_a = st_ref[:, 0:HQ]
            l_a = st_ref[:, HQ:2 * HQ]
            m_b = rbuf_st[k, :, 0:HQ]
            l_b = rbuf_st[k, :, HQ:2 * HQ]
            m_n = jnp.maximum(m_a, m_b)
            a_a = jnp.exp(m_a - m_n)
            a_b = jnp.exp(m_b - m_n)
            st_ref[:, 0:HQ] = m_n
            st_ref[:, HQ:2 * HQ] = l_a * a_a + l_b * a_b
            for h in range(HQ):
                acc_ref[:, h * DH:(h + 1) * DH] = (
                    acc_ref[:, h * DH:(h + 1) * DH] * a_a[:, h:h + 1]
                    + rbuf_o[k, :, h * DH:(h + 1) * DH] * a_b[:, h:h + 1])

        l_fin = st_ref[:, HQ:2 * HQ]
        o_norm = jnp.concatenate(
            [acc_ref[:, h * DH:(h + 1) * DH] / l_fin[:, h:h + 1]
             for h in range(HQ)], axis=1)
        out_ref[0] = jnp.dot(o_norm, wo_ref[...],
                             preferred_element_type=jnp.float32)

    return pl.pallas_call(
        body,
        out_shape=jax.ShapeDtypeStruct((1, SQ, D), jnp.float32),
        in_specs=[pl.BlockSpec(memory_space=pltpu.VMEM)] * 5,
        out_specs=pl.BlockSpec(memory_space=pltpu.VMEM),
        scratch_shapes=[
            pltpu.VMEM((SQ, D), jnp.float32),
            pltpu.VMEM((SQ, 2 * HQ), jnp.float32),
            pltpu.VMEM((N_STAGES, SQ, D), jnp.float32),
            pltpu.VMEM((N_STAGES, SQ, 2 * HQ), jnp.float32),
            pltpu.SemaphoreType.DMA((N_STAGES,)),
            pltpu.SemaphoreType.DMA((N_STAGES,)),
            pltpu.SemaphoreType.DMA((N_STAGES,)),
            pltpu.SemaphoreType.DMA((N_STAGES,)),
        ],
        compiler_params=pltpu.CompilerParams(collective_id=0),
    )(x, Wq, Wo, K_ext, V_ext)


# baseline (device time: 128676 ns/iter reference)
import jax
import jax.numpy as jnp
from jax import lax
from jax.experimental import pallas as pl
from jax.experimental.pallas import tpu as pltpu

N_DEV = 16
N_STAGES = 4
SQ = 256
HQ = 8
DH = 128
D = HQ * DH
SCALE = 0.08838834764831843


def kernel(x, Wq, Wo, K_ext, V_ext):
    def body(x_ref, wq_ref, wo_ref, k_hbm, v_hbm, out_ref,
             acc_ref, st_ref, rbuf_o, rbuf_st, kh_buf, vh_buf,
             o_send, o_recv, st_send, st_recv, kv_sems):
        my = lax.axis_index("i")

        barrier = pltpu.get_barrier_semaphore()
        for k in range(N_STAGES):
            partner = my ^ (1 << k)
            pl.semaphore_signal(barrier, inc=1, device_id=(partner,),
                                device_id_type=pl.DeviceIdType.MESH)
        pl.semaphore_wait(barrier, N_STAGES)

        q = jnp.dot(x_ref[0], wq_ref[...],
                    preferred_element_type=jnp.float32) * SCALE
        for h in range(HQ):
            k_cp = pltpu.make_async_copy(
                k_hbm.at[0, :, h, :], kh_buf, kv_sems.at[0])
            v_cp = pltpu.make_async_copy(
                v_hbm.at[0, :, h, :], vh_buf, kv_sems.at[1])
            k_cp.start()
            v_cp.start()
            k_cp.wait()
            v_cp.wait()
            qh = q[:, h * DH:(h + 1) * DH]
            kh = kh_buf[...]
            vh = vh_buf[...]
            s = lax.dot_general(qh, kh, (((1,), (1,)), ((), ())),
                                preferred_element_type=jnp.float32)
            m = jnp.max(s, axis=1, keepdims=True)
            p = jnp.exp(s - m)
            l = jnp.sum(p, axis=1, keepdims=True)
            o = jnp.dot(p, vh, preferred_element_type=jnp.float32)
            acc_ref[:, h * DH:(h + 1) * DH] = o
            st_ref[:, h:h + 1] = m
            st_ref[:, HQ + h:HQ + h + 1] = l

        for k in range(N_STAGES):
            partner = my ^ (1 << k)
            o_rdma = pltpu.make_async_remote_copy(
                src_ref=acc_ref, dst_ref=rbuf_o.at[k],
                send_sem=o_send.at[k], recv_sem=o_recv.at[k],
                device_id=(partner,), device_id_type=pl.DeviceIdType.MESH)
            st_rdma = pltpu.make_async_remote_copy(
                src_ref=st_ref, dst_ref=rbuf_st.at[k],
                send_sem=st_send.at[k], recv_sem=st_recv.at[k],
                device_id=(partner,), device_id_type=pl.DeviceIdType.MESH)
            o_rdma.start()
            st_rdma.start()
            o_rdma.wait()
            st_rdma.wait()

            m_a = st_ref[:, 0:HQ]
            l_a = st_ref[:, HQ:2 * HQ]
            m_b = rbuf_st[k, :, 0:HQ]
            l_b = rbuf_st[k, :, HQ:2 * HQ]
            m_n = jnp.maximum(m_a, m_b)
            a_a = jnp.exp(m_a - m_n)
            a_b = jnp.exp(m_b - m_n)
            st_ref[:, 0:HQ] = m_n
            st_ref[:, HQ:2 * HQ] = l_a * a_a + l_b * a_b
            for h in range(HQ):
                acc_ref[:, h * DH:(h + 1) * DH] = (
                    acc_ref[:, h * DH:(h + 1) * DH] * a_a[:, h:h + 1]
                    + rbuf_o[k, :, h * DH:(h + 1) * DH] * a_b[:, h:h + 1])

        l_fin = st_ref[:, HQ:2 * HQ]
        o_norm = jnp.concatenate(
            [acc_ref[:, h * DH:(h + 1) * DH] / l_fin[:, h:h + 1]
             for h in range(HQ)], axis=1)
        out_ref[0] = jnp.dot(o_norm, wo_ref[...],
                             preferred_element_type=jnp.float32)

    return pl.pallas_call(
        body,
        out_shape=jax.ShapeDtypeStruct((1, SQ, D), jnp.float32),
        in_specs=[pl.BlockSpec(memory_space=pltpu.VMEM)] * 3
        + [pl.BlockSpec(memory_space=pltpu.MemorySpace.HBM)] * 2,
        out_specs=pl.BlockSpec(memory_space=pltpu.VMEM),
        scratch_shapes=[
            pltpu.VMEM((SQ, D), jnp.float32),
            pltpu.VMEM((SQ, 2 * HQ), jnp.float32),
            pltpu.VMEM((N_STAGES, SQ, D), jnp.float32),
            pltpu.VMEM((N_STAGES, SQ, 2 * HQ), jnp.float32),
            pltpu.VMEM((4096, DH), jnp.float32),
            pltpu.VMEM((4096, DH), jnp.float32),
            pltpu.SemaphoreType.DMA((N_STAGES,)),
            pltpu.SemaphoreType.DMA((N_STAGES,)),
            pltpu.SemaphoreType.DMA((N_STAGES,)),
            pltpu.SemaphoreType.DMA((N_STAGES,)),
            pltpu.SemaphoreType.DMA((2,)),
        ],
        compiler_params=pltpu.CompilerParams(
            collective_id=0, vmem_limit_bytes=100 * 1024 * 1024),
    )(x, Wq, Wo, K_ext, V_ext)


# device time: 113635 ns/iter; 1.1324x vs baseline; 1.1324x over previous
import jax
import jax.numpy as jnp
from jax import lax
from jax.experimental import pallas as pl
from jax.experimental.pallas import tpu as pltpu

N_DEV = 16
N_STAGES = 4
SQ = 256
HQ = 8
DH = 128
D = HQ * DH
SCALE = 0.08838834764831843


def kernel(x, Wq, Wo, K_ext, V_ext):
    def body(x_ref, wq_ref, wo_ref, k_hbm, v_hbm, out_ref,
             acc_ref, st_ref, rbuf_o, rbuf_st, kh_buf, vh_buf,
             o_send, o_recv, st_send, st_recv, kv_sems):
        my = lax.axis_index("i")

        barrier = pltpu.get_barrier_semaphore()
        for k in range(N_STAGES):
            partner = my ^ (1 << k)
            pl.semaphore_signal(barrier, inc=1, device_id=(partner,),
                                device_id_type=pl.DeviceIdType.MESH)
        pl.semaphore_wait(barrier, N_STAGES)

        def kv_copy(h, slot):
            k_cp = pltpu.make_async_copy(
                k_hbm.at[0, :, h, :], kh_buf.at[slot], kv_sems.at[slot, 0])
            v_cp = pltpu.make_async_copy(
                v_hbm.at[0, :, h, :], vh_buf.at[slot], kv_sems.at[slot, 1])
            k_cp.start()
            v_cp.start()
            return k_cp, v_cp

        pending = kv_copy(0, 0)
        q = jnp.dot(x_ref[0], wq_ref[...],
                    preferred_element_type=jnp.float32) * SCALE
        for h in range(HQ):
            slot = h % 2
            pending[0].wait()
            pending[1].wait()
            if h + 1 < HQ:
                pending = kv_copy(h + 1, (h + 1) % 2)
            qh = q[:, h * DH:(h + 1) * DH]
            kh = kh_buf[slot]
            vh = vh_buf[slot]
            s = lax.dot_general(qh, kh, (((1,), (1,)), ((), ())),
                                preferred_element_type=jnp.float32)
            m = jnp.max(s, axis=1, keepdims=True)
            p = jnp.exp(s - m)
            l = jnp.sum(p, axis=1, keepdims=True)
            o = jnp.dot(p, vh, preferred_element_type=jnp.float32)
            acc_ref[:, h * DH:(h + 1) * DH] = o
            st_ref[:, h:h + 1] = m
            st_ref[:, HQ + h:HQ + h + 1] = l

        for k in range(N_STAGES):
            partner = my ^ (1 << k)
            o_rdma = pltpu.make_async_remote_copy(
                src_ref=acc_ref, dst_ref=rbuf_o.at[k],
                send_sem=o_send.at[k], recv_sem=o_recv.at[k],
                device_id=(partner,), device_id_type=pl.DeviceIdType.MESH)
            st_rdma = pltpu.make_async_remote_copy(
                src_ref=st_ref, dst_ref=rbuf_st.at[k],
                send_sem=st_send.at[k], recv_sem=st_recv.at[k],
                device_id=(partner,), device_id_type=pl.DeviceIdType.MESH)
            o_rdma.start()
            st_rdma.start()
            o_rdma.wait()
            st_rdma.wait()

            m_a = st_ref[:, 0:HQ]
            l_a = st_ref[:, HQ:2 * HQ]
            m_b = rbuf_st[k, :, 0:HQ]
            l_b = rbuf_st[k, :, HQ:2 * HQ]
            m_n = jnp.maximum(m_a, m_b)
            a_a = jnp.exp(m_a - m_n)
            a_b = jnp.exp(m_b - m_n)
            st_ref[:, 0:HQ] = m_n
            st_ref[:, HQ:2 * HQ] = l_a * a_a + l_b * a_b
            for h in range(HQ):
                acc_ref[:, h * DH:(h + 1) * DH] = (
                    acc_ref[:, h * DH:(h + 1) * DH] * a_a[:, h:h + 1]
                    + rbuf_o[k, :, h * DH:(h + 1) * DH] * a_b[:, h:h + 1])

        l_fin = st_ref[:, HQ:2 * HQ]
        o_norm = jnp.concatenate(
            [acc_ref[:, h * DH:(h + 1) * DH] / l_fin[:, h:h + 1]
             for h in range(HQ)], axis=1)
        out_ref[0] = jnp.dot(o_norm, wo_ref[...],
                             preferred_element_type=jnp.float32)

    return pl.pallas_call(
        body,
        out_shape=jax.ShapeDtypeStruct((1, SQ, D), jnp.float32),
        in_specs=[pl.BlockSpec(memory_space=pltpu.VMEM)] * 3
        + [pl.BlockSpec(memory_space=pltpu.MemorySpace.HBM)] * 2,
        out_specs=pl.BlockSpec(memory_space=pltpu.VMEM),
        scratch_shapes=[
            pltpu.VMEM((SQ, D), jnp.float32),
            pltpu.VMEM((SQ, 2 * HQ), jnp.float32),
            pltpu.VMEM((N_STAGES, SQ, D), jnp.float32),
            pltpu.VMEM((N_STAGES, SQ, 2 * HQ), jnp.float32),
            pltpu.VMEM((2, 4096, DH), jnp.float32),
            pltpu.VMEM((2, 4096, DH), jnp.float32),
            pltpu.SemaphoreType.DMA((N_STAGES,)),
            pltpu.SemaphoreType.DMA((N_STAGES,)),
            pltpu.SemaphoreType.DMA((N_STAGES,)),
            pltpu.SemaphoreType.DMA((N_STAGES,)),
            pltpu.SemaphoreType.DMA((2, 2)),
        ],
        compiler_params=pltpu.CompilerParams(
            collective_id=0, vmem_limit_bytes=100 * 1024 * 1024),
    )(x, Wq, Wo, K_ext, V_ext)


# device time: 76792 ns/iter; 1.6756x vs baseline; 1.4798x over previous
import jax
import jax.numpy as jnp
from jax import lax
from jax.experimental import pallas as pl
from jax.experimental.pallas import tpu as pltpu

N_DEV = 16
N_STAGES = 4
SQ = 256
HQ = 8
DH = 128
D = HQ * DH
SCALE = 0.08838834764831843


def kernel(x, Wq, Wo, K_ext, V_ext):
    def body(x_ref, wq_ref, wo_ref, k_hbm, v_hbm, out_ref,
             acc_ref, st_ref, rbuf_o, rbuf_st, kh_buf, vh_buf,
             rs_o_send, rs_o_recv, rs_st_send, rs_st_recv,
             ag_send, ag_recv, kv_sems):
        my = lax.axis_index("i")

        barrier = pltpu.get_barrier_semaphore()
        for k in range(N_STAGES):
            partner = my ^ (1 << k)
            pl.semaphore_signal(barrier, inc=1, device_id=(partner,),
                                device_id_type=pl.DeviceIdType.MESH)
        pl.semaphore_wait(barrier, N_STAGES)

        def kv_copy(h, slot):
            k_cp = pltpu.make_async_copy(
                k_hbm.at[0, :, h, :], kh_buf.at[slot], kv_sems.at[slot, 0])
            v_cp = pltpu.make_async_copy(
                v_hbm.at[0, :, h, :], vh_buf.at[slot], kv_sems.at[slot, 1])
            k_cp.start()
            v_cp.start()
            return k_cp, v_cp

        pending = kv_copy(0, 0)
        q = jnp.dot(x_ref[0], wq_ref[...],
                    preferred_element_type=jnp.float32) * SCALE
        for h in range(HQ):
            slot = h % 2
            pending[0].wait()
            pending[1].wait()
            if h + 1 < HQ:
                pending = kv_copy(h + 1, (h + 1) % 2)
            qh = q[:, h * DH:(h + 1) * DH]
            kh = kh_buf[slot]
            vh = vh_buf[slot]
            s = lax.dot_general(qh, kh, (((1,), (1,)), ((), ())),
                                preferred_element_type=jnp.float32)
            m = jnp.max(s, axis=1, keepdims=True)
            p = jnp.exp(s - m)
            l = jnp.sum(p, axis=1, keepdims=True)
            o = jnp.dot(p, vh, preferred_element_type=jnp.float32)
            acc_ref[:, h * DH:(h + 1) * DH] = o
            st_ref[:, h:h + 1] = m
            st_ref[:, HQ + h:HQ + h + 1] = l

        offs = [jnp.int32(0)]
        for k in range(N_STAGES):
            bit = (my >> k) & 1
            offs.append(offs[-1] + bit * (SQ >> (k + 1)))

        for k in range(N_STAGES):
            half = SQ >> (k + 1)
            bit = (my >> k) & 1
            send_off = offs[k] + (1 - bit) * half
            keep_off = offs[k + 1]
            o_rdma = pltpu.make_async_remote_copy(
                src_ref=acc_ref.at[pl.ds(send_off, half), :],
                dst_ref=rbuf_o.at[k, pl.ds(0, half), :],
                send_sem=rs_o_send.at[k], recv_sem=rs_o_recv.at[k],
                device_id=(my ^ (1 << k),),
                device_id_type=pl.DeviceIdType.MESH)
            st_rdma = pltpu.make_async_remote_copy(
                src_ref=st_ref.at[pl.ds(send_off, half), :],
                dst_ref=rbuf_st.at[k, pl.ds(0, half), :],
                send_sem=rs_st_send.at[k], recv_sem=rs_st_recv.at[k],
                device_id=(my ^ (1 << k),),
                device_id_type=pl.DeviceIdType.MESH)
            o_rdma.start()
            st_rdma.start()
            o_rdma.wait()
            st_rdma.wait()

            m_a = st_ref[pl.ds(keep_off, half), 0:HQ]
            l_a = st_ref[pl.ds(keep_off, half), HQ:2 * HQ]
            m_b = rbuf_st[k, pl.ds(0, half), 0:HQ]
            l_b = rbuf_st[k, pl.ds(0, half), HQ:2 * HQ]
            m_n = jnp.maximum(m_a, m_b)
            a_a = jnp.exp(m_a - m_n)
            a_b = jnp.exp(m_b - m_n)
            st_ref[pl.ds(keep_off, half), 0:HQ] = m_n
            st_ref[pl.ds(keep_off, half), HQ:2 * HQ] = l_a * a_a + l_b * a_b
            for h in range(HQ):
                acc_ref[pl.ds(keep_off, half), h * DH:(h + 1) * DH] = (
                    acc_ref[pl.ds(keep_off, half), h * DH:(h + 1) * DH]
                    * a_a[:, h:h + 1]
                    + rbuf_o[k, pl.ds(0, half), h * DH:(h + 1) * DH]
                    * a_b[:, h:h + 1])

        nrows = SQ >> N_STAGES
        my_off = offs[N_STAGES]
        l_fin = st_ref[pl.ds(my_off, nrows), HQ:2 * HQ]
        o_norm = jnp.concatenate(
            [acc_ref[pl.ds(my_off, nrows), h * DH:(h + 1) * DH]
             / l_fin[:, h:h + 1] for h in range(HQ)], axis=1)
        out_ref[0, pl.ds(my_off, nrows), :] = jnp.dot(
            o_norm, wo_ref[...], preferred_element_type=jnp.float32)

        for k in reversed(range(N_STAGES)):
            bs = SQ >> (k + 1)
            ag = pltpu.make_async_remote_copy(
                src_ref=out_ref.at[0, pl.ds(offs[k + 1], bs), :],
                dst_ref=out_ref.at[0, pl.ds(offs[k + 1], bs), :],
                send_sem=ag_send.at[k], recv_sem=ag_recv.at[k],
                device_id=(my ^ (1 << k),),
                device_id_type=pl.DeviceIdType.MESH)
            ag.start()
            ag.wait()

    return pl.pallas_call(
        body,
        out_shape=jax.ShapeDtypeStruct((1, SQ, D), jnp.float32),
        in_specs=[pl.BlockSpec(memory_space=pltpu.VMEM)] * 3
        + [pl.BlockSpec(memory_space=pltpu.MemorySpace.HBM)] * 2,
        out_specs=pl.BlockSpec(memory_space=pltpu.VMEM),
        scratch_shapes=[
            pltpu.VMEM((SQ, D), jnp.float32),
            pltpu.VMEM((SQ, 2 * HQ), jnp.float32),
            pltpu.VMEM((N_STAGES, SQ // 2, D), jnp.float32),
            pltpu.VMEM((N_STAGES, SQ // 2, 2 * HQ), jnp.float32),
            pltpu.VMEM((2, 4096, DH), jnp.float32),
            pltpu.VMEM((2, 4096, DH), jnp.float32),
            pltpu.SemaphoreType.DMA((N_STAGES,)),
            pltpu.SemaphoreType.DMA((N_STAGES,)),
            pltpu.SemaphoreType.DMA((N_STAGES,)),
            pltpu.SemaphoreType.DMA((N_STAGES,)),
            pltpu.SemaphoreType.DMA((N_STAGES,)),
            pltpu.SemaphoreType.DMA((N_STAGES,)),
            pltpu.SemaphoreType.DMA((2, 2)),
        ],
        compiler_params=pltpu.CompilerParams(
            collective_id=0, vmem_limit_bytes=100 * 1024 * 1024),
    )(x, Wq, Wo, K_ext, V_ext)


# device time: 34693 ns/iter; 3.7090x vs baseline; 2.2135x over previous
import os

import jax
import jax.numpy as jnp
from jax import lax
from jax.experimental import pallas as pl
from jax.experimental.pallas import tpu as pltpu

_PROBE = os.environ.get("PROBE", "")

N_DEV = 16
N_STAGES = 4
SQ = 256
HQ = 8
DH = 128
D = HQ * DH
SCALE = 0.08838834764831843


def kernel(x, Wq, Wo, K_ext, V_ext):
    def body(x_ref, wq_ref, wo_ref, k_hbm, v_hbm, out_ref,
             acc_ref, st_ref, rbuf_o, rbuf_st, kh_buf, vh_buf,
             rs_o_send, rs_o_recv, rs_st_send, rs_st_recv,
             ag_send, ag_recv, kv_sems):
        my = lax.axis_index("i")

        barrier = pltpu.get_barrier_semaphore()
        for k in range(N_STAGES):
            partner = my ^ (1 << k)
            pl.semaphore_signal(barrier, inc=1, device_id=(partner,),
                                device_id_type=pl.DeviceIdType.MESH)
        pl.semaphore_wait(barrier, N_STAGES)

        def kv_copy(h, slot):
            k_cp = pltpu.make_async_copy(
                k_hbm.at[0, :, h, :], kh_buf.at[slot], kv_sems.at[slot, 0])
            v_cp = pltpu.make_async_copy(
                v_hbm.at[0, :, h, :], vh_buf.at[slot], kv_sems.at[slot, 1])
            k_cp.start()
            v_cp.start()
            return k_cp, v_cp

        pending = kv_copy(0, 0)
        q = jnp.dot(x_ref[0], wq_ref[...],
                    preferred_element_type=jnp.float32) * SCALE
        for h in range(HQ):
            slot = h % 2
            pending[0].wait()
            pending[1].wait()
            if h + 1 < HQ:
                pending = kv_copy(h + 1, (h + 1) % 2)
            qh = q[:, h * DH:(h + 1) * DH]
            kh = kh_buf[slot]
            vh = vh_buf[slot]
            s = lax.dot_general(qh, kh, (((1,), (1,)), ((), ())),
                                preferred_element_type=jnp.float32)
            m = jnp.max(s, axis=1, keepdims=True)
            p = jnp.exp(s - m)
            l = jnp.sum(p, axis=1, keepdims=True)
            o = jnp.dot(p, vh, preferred_element_type=jnp.float32)
            acc_ref[:, h * DH:(h + 1) * DH] = o
            st_ref[:, h:h + 1] = m
            st_ref[:, HQ + h:HQ + h + 1] = l

        offs = [jnp.int32(0)]
        for k in range(N_STAGES):
            bit = (my >> k) & 1
            offs.append(offs[-1] + bit * (SQ >> (k + 1)))

        for k in range(N_STAGES):
            half = SQ >> (k + 1)
            bit = (my >> k) & 1
            send_off = offs[k] + (1 - bit) * half
            keep_off = offs[k + 1]
            o_rdma = pltpu.make_async_remote_copy(
                src_ref=acc_ref.at[pl.ds(send_off, half), :],
                dst_ref=rbuf_o.at[k, pl.ds(0, half), :],
                send_sem=rs_o_send.at[k], recv_sem=rs_o_recv.at[k],
                device_id=(my ^ (1 << k),),
                device_id_type=pl.DeviceIdType.MESH)
            st_rdma = pltpu.make_async_remote_copy(
                src_ref=st_ref.at[pl.ds(send_off, half), :],
                dst_ref=rbuf_st.at[k, pl.ds(0, half), :],
                send_sem=rs_st_send.at[k], recv_sem=rs_st_recv.at[k],
                device_id=(my ^ (1 << k),),
                device_id_type=pl.DeviceIdType.MESH)
            if _PROBE != "nocomm":
                o_rdma.start()
                st_rdma.start()
                o_rdma.wait()
                st_rdma.wait()

            m_a = st_ref[pl.ds(keep_off, half), 0:HQ]
            l_a = st_ref[pl.ds(keep_off, half), HQ:2 * HQ]
            m_b = rbuf_st[k, pl.ds(0, half), 0:HQ]
            l_b = rbuf_st[k, pl.ds(0, half), HQ:2 * HQ]
            m_n = jnp.maximum(m_a, m_b)
            a_a = jnp.exp(m_a - m_n)
            a_b = jnp.exp(m_b - m_n)
            st_ref[pl.ds(keep_off, half), 0:HQ] = m_n
            st_ref[pl.ds(keep_off, half), HQ:2 * HQ] = l_a * a_a + l_b * a_b
            for h in range(HQ):
                acc_ref[pl.ds(keep_off, half), h * DH:(h + 1) * DH] = (
                    acc_ref[pl.ds(keep_off, half), h * DH:(h + 1) * DH]
                    * a_a[:, h:h + 1]
                    + rbuf_o[k, pl.ds(0, half), h * DH:(h + 1) * DH]
                    * a_b[:, h:h + 1])

        nrows = SQ >> N_STAGES
        my_off = offs[N_STAGES]
        l_fin = st_ref[pl.ds(my_off, nrows), HQ:2 * HQ]
        o_norm = jnp.concatenate(
            [acc_ref[pl.ds(my_off, nrows), h * DH:(h + 1) * DH]
             / l_fin[:, h:h + 1] for h in range(HQ)], axis=1)
        out_ref[0, pl.ds(my_off, nrows), :] = jnp.dot(
            o_norm, wo_ref[...], preferred_element_type=jnp.float32)

        for k in reversed(range(N_STAGES)):
            bs = SQ >> (k + 1)
            ag = pltpu.make_async_remote_copy(
                src_ref=out_ref.at[0, pl.ds(offs[k + 1], bs), :],
                dst_ref=out_ref.at[0, pl.ds(offs[k + 1], bs), :],
                send_sem=ag_send.at[k], recv_sem=ag_recv.at[k],
                device_id=(my ^ (1 << k),),
                device_id_type=pl.DeviceIdType.MESH)
            if _PROBE != "nocomm":
                ag.start()
                ag.wait()

    return pl.pallas_call(
        body,
        out_shape=jax.ShapeDtypeStruct((1, SQ, D), jnp.float32),
        in_specs=[pl.BlockSpec(memory_space=pltpu.VMEM)] * 3
        + [pl.BlockSpec(memory_space=pltpu.MemorySpace.HBM)] * 2,
        out_specs=pl.BlockSpec(memory_space=pltpu.VMEM),
        scratch_shapes=[
            pltpu.VMEM((SQ, D), jnp.float32),
            pltpu.VMEM((SQ, 2 * HQ), jnp.float32),
            pltpu.VMEM((N_STAGES, SQ // 2, D), jnp.float32),
            pltpu.VMEM((N_STAGES, SQ // 2, 2 * HQ), jnp.float32),
            pltpu.VMEM((2, 4096, DH), jnp.float32),
            pltpu.VMEM((2, 4096, DH), jnp.float32),
            pltpu.SemaphoreType.DMA((N_STAGES,)),
            pltpu.SemaphoreType.DMA((N_STAGES,)),
            pltpu.SemaphoreType.DMA((N_STAGES,)),
            pltpu.SemaphoreType.DMA((N_STAGES,)),
            pltpu.SemaphoreType.DMA((N_STAGES,)),
            pltpu.SemaphoreType.DMA((N_STAGES,)),
            pltpu.SemaphoreType.DMA((2, 2)),
        ],
        compiler_params=pltpu.CompilerParams(
            collective_id=0, vmem_limit_bytes=100 * 1024 * 1024),
    )(x, Wq, Wo, K_ext, V_ext)
